# baseline (speedup 1.0000x reference)
import functools

import jax
import jax.numpy as jnp
from jax import lax
from jax.experimental import pallas as pl
from jax.experimental.pallas import tpu as pltpu
from jax.experimental.pallas import tpu_sc as plsc

_NC, _NS, _L = 2, 16, 16
_NW = _NC * _NS

_B, _N, _Q = 4, 10000, 512
_QPW = (_B * _Q) // _NW
_SLOTS = _Q // _QPW
_NCH = _N // _L
_GROUP = 25
_NGRP = _NCH // _GROUP
_NQB = 4
_S = 128
_S2 = 32
_S2PAD = _S2 + 4
_OUTW = 16

_BIG = 3e38


def _knn_body(noisy_hbm, clean_hbm, idx_hbm, out_hbm,
              px, py, pz, cx, cy, cz, sqn, sqc, idx_v,
              cand_d, cand_i, b2d, b2i, out_v):
    wid = lax.axis_index("s") * _NC + lax.axis_index("c")
    b = wid // _SLOTS
    slot = wid % _SLOTS
    base = b * 3 * _N
    pltpu.sync_copy(noisy_hbm.at[pl.ds(base, _N)], px)
    pltpu.sync_copy(noisy_hbm.at[pl.ds(base + _N, _N)], py)
    pltpu.sync_copy(noisy_hbm.at[pl.ds(base + 2 * _N, _N)], pz)
    pltpu.sync_copy(clean_hbm.at[pl.ds(base, _N)], cx)
    pltpu.sync_copy(clean_hbm.at[pl.ds(base + _N, _N)], cy)
    pltpu.sync_copy(clean_hbm.at[pl.ds(base + 2 * _N, _N)], cz)
    pltpu.sync_copy(idx_hbm.at[pl.ds(slot * _QPW, _QPW)],
                    idx_v.at[pl.ds(0, _QPW)])

    def sqinit(c, _):
        sl = pl.ds(c * _L, _L)
        vx = px[sl]
        vy = py[sl]
        vz = pz[sl]
        sqn[sl] = vx * vx + vy * vy + vz * vz
        vx = cx[sl]
        vy = cy[sl]
        vz = cz[sl]
        sqc[sl] = vx * vx + vy * vy + vz * vz
        return _
    lax.fori_loop(0, _NCH, sqinit, 0)

    iota = lax.iota(jnp.int32, _L)
    zf = jnp.zeros((_L,), jnp.float32)
    zi = jnp.zeros((_L,), jnp.int32)
    cbase = [qq * (_S * _L) for qq in range(_NQB)]
    climits = [cbase[qq] + (_S - 1) * _L + iota for qq in range(_NQB)]

    def _shuf(v, s):
        return v.at[iota ^ s].get(mode="promise_in_bounds")

    def vmaxs(v):
        for s in (8, 4, 2, 1):
            v = jnp.maximum(v, _shuf(v, s))
        return v

    def vadds(v):
        for s in (8, 4, 2, 1):
            v = v + _shuf(v, s)
        return v

    def scan4(xr, yr, zr, sq, tqx, tqy, tqz, depth):
        def grp(g, carry):
            m1s, m2s, offp, tbs = carry
            m1s, m2s, offp, tbs = list(m1s), list(m2s), list(offp), list(tbs)
            for k in range(_GROUP):
                c = g * _GROUP + k
                sl = pl.ds(c * _L, _L)
                vx = xr[sl]
                vy = yr[sl]
                vz = zr[sl]
                vs = sq[sl]
                ci = c * _L + iota
                for qq in range(_NQB):
                    d = (vs - vx * tqx[qq]
                         - vy * tqy[qq] - vz * tqz[qq])
                    if depth == 2 and k % 2 == 1:
                        m2s[qq] = jnp.minimum(m2s[qq], d)
                    else:
                        m1s[qq] = jnp.minimum(m1s[qq], d)
                    msk = d <= tbs[qq]
                    pos = jnp.minimum(offp[qq], climits[qq])
                    plsc.store_scatter(cand_d, [pos], d, mask=msk)
                    plsc.store_scatter(cand_i, [pos], ci, mask=msk)
                    offp[qq] = offp[qq] + jnp.where(msk, 16, 0)
            for qq in range(_NQB):
                tbs[qq] = vmaxs(jnp.maximum(m1s[qq], m2s[qq])
                                if depth == 2 else m1s[qq])
            return tuple(m1s), tuple(m2s), tuple(offp), tuple(tbs)

        big = jnp.full((_L,), _BIG, jnp.float32)
        init = ((big,) * _NQB, (big,) * _NQB,
                tuple(cbase[qq] + iota for qq in range(_NQB)), (big,) * _NQB)
        m1s, m2s, offp, _ = lax.fori_loop(0, _NGRP, grp, init)
        if depth == 2:
            msrc = tuple(jnp.maximum(m1s[qq], m2s[qq]) for qq in range(_NQB))
        else:
            msrc = m1s
        offls = tuple((offp[qq] - cbase[qq] - iota) >> 4
                      for qq in range(_NQB))
        return msrc, offls

    def finish4(msrcs, offls, xr, yr, zr, K):
        texs = [vmaxs(msrcs[qq]) for qq in range(_NQB)]
        rmax = jnp.maximum(jnp.maximum(offls[0], offls[1]),
                           jnp.maximum(offls[2], offls[3]))
        rows = jnp.minimum(jnp.max(rmax), _S)

        def shrink(r, off2s):
            off2s = list(off2s)
            for qq in range(_NQB):
                sl = pl.ds(qq * (_S * _L) + r * _L, _L)
                d = cand_d[sl]
                vi = cand_i[sl]
                msk = (d <= texs[qq]) & (offls[qq] > r)
                pos = (qq * (_S2PAD * _L)
                       + jnp.minimum(off2s[qq], _S2 - 1) * _L + iota)
                plsc.store_scatter(b2d, [pos], d, mask=msk)
                plsc.store_scatter(b2i, [pos], vi, mask=msk)
                off2s[qq] = off2s[qq] + msk.astype(jnp.int32)
            return tuple(off2s)
        off2s = lax.fori_loop(0, rows, shrink, (zi,) * _NQB)
        off2s = [jnp.minimum(o, _S2) for o in off2s]
        rows2 = jnp.max(jnp.maximum(jnp.maximum(off2s[0], off2s[1]),
                                    jnp.maximum(off2s[2], off2s[3])))

        big = jnp.full((_L,), _BIG, jnp.float32)
        if K == 32:
            def fold(r, carry):
                new = []
                for qq in range(_NQB):
                    lok, lov, hik, hiv = carry[qq]
                    sl = pl.ds(qq * (_S2PAD * _L) + r * _L, _L)
                    kd = jnp.where(off2s[qq] > r, b2d[sl], big)
                    sk, sv = plsc.sort_key_val(kd, b2i[sl])
                    rsk = _shuf(sk, 15)
                    rsv = _shuf(sv, 15)
                    m = hik <= rsk
                    wk = jnp.where(m, hik, rsk)
                    wv = jnp.where(m, hiv, rsv)
                    wk, wv = plsc.sort_key_val(wk, wv)
                    rwk = _shuf(wk, 15)
                    rwv = _shuf(wv, 15)
                    m2 = lok <= rwk
                    nlk = jnp.where(m2, lok, rwk)
                    nlv = jnp.where(m2, lov, rwv)
                    nhk = jnp.where(m2, rwk, lok)
                    nhv = jnp.where(m2, rwv, lov)
                    nlk, nlv = plsc.sort_key_val(nlk, nlv)
                    nhk, nhv = plsc.sort_key_val(nhk, nhv)
                    new.append((nlk, nlv, nhk, nhv))
                return tuple(new)
            st = lax.fori_loop(0, rows2, fold, ((big, zi, big, zi),) * _NQB)
            sums = []
            for qq in range(_NQB):
                _, lov, _, hiv = st[qq]
                gx = plsc.load_gather(xr, [lov]) + plsc.load_gather(xr, [hiv])
                gy = plsc.load_gather(yr, [lov]) + plsc.load_gather(yr, [hiv])
                gz = plsc.load_gather(zr, [lov]) + plsc.load_gather(zr, [hiv])
                sums.append((vadds(gx), vadds(gy), vadds(gz)))
            return sums
        else:
            def fold(r, carry):
                new = []
                for qq in range(_NQB):
                    lok, lov = carry[qq]
                    sl = pl.ds(qq * (_S2PAD * _L) + r * _L, _L)
                    kd = jnp.where(off2s[qq] > r, b2d[sl], big)
                    sk, sv = plsc.sort_key_val(kd, b2i[sl])
                    rsk = _shuf(sk, 15)
                    rsv = _shuf(sv, 15)
                    m = lok <= rsk
                    wk = jnp.where(m, lok, rsk)
                    wv = jnp.where(m, lov, rsv)
                    lok, lov = plsc.sort_key_val(wk, wv)
                    new.append((lok, lov))
                return tuple(new)
            st = lax.fori_loop(0, rows2, fold, ((big, zi),) * _NQB)
            mk = iota < K
            sums = []
            for qq in range(_NQB):
                _, lov = st[qq]
                gx = plsc.load_gather(xr, [lov], mask=mk)
                gy = plsc.load_gather(yr, [lov], mask=mk)
                gz = plsc.load_gather(zr, [lov], mask=mk)
                sums.append((vadds(jnp.where(mk, gx, zf)),
                             vadds(jnp.where(mk, gy, zf)),
                             vadds(jnp.where(mk, gz, zf))))
            return sums

    def qgroup(jg, carry):
        j0 = jg * _NQB
        qxs, qys, qzs = [], [], []
        for qq in range(_NQB):
            qidx = plsc.load_gather(
                idx_v, [jnp.full((_L,), j0 + qq, jnp.int32)])
            qxs.append(plsc.load_gather(px, [qidx]))
            qys.append(plsc.load_gather(py, [qidx]))
            qzs.append(plsc.load_gather(pz, [qidx]))
        tqx = [qxs[qq] + qxs[qq] for qq in range(_NQB)]
        tqy = [qys[qq] + qys[qq] for qq in range(_NQB)]
        tqz = [qzs[qq] + qzs[qq] for qq in range(_NQB)]
        msrc, offls = scan4(px, py, pz, sqn, tqx, tqy, tqz, 2)
        nsum = finish4(msrc, offls, px, py, pz, 32)
        msrc, offls = scan4(cx, cy, cz, sqc, tqx, tqy, tqz, 1)
        esum = finish4(msrc, offls, cx, cy, cz, 4)
        for qq in range(_NQB):
            out = jnp.where(iota == 0, qxs[qq], zf)
            out = jnp.where(iota == 1, qys[qq], out)
            out = jnp.where(iota == 2, qzs[qq], out)
            out = jnp.where(iota == 3, nsum[qq][0], out)
            out = jnp.where(iota == 4, nsum[qq][1], out)
            out = jnp.where(iota == 5, nsum[qq][2], out)
            out = jnp.where(iota == 6, esum[qq][0], out)
            out = jnp.where(iota == 7, esum[qq][1], out)
            out = jnp.where(iota == 8, esum[qq][2], out)
            out_v[pl.ds((j0 + qq) * _OUTW, _OUTW)] = out
        return carry
    lax.fori_loop(0, _QPW // _NQB, qgroup, 0)
    pltpu.sync_copy(out_v, out_hbm.at[pl.ds(wid * _QPW * _OUTW, _QPW * _OUTW)])


_knn_call = functools.partial(
    pl.kernel,
    out_type=jax.ShapeDtypeStruct((_NW * _QPW * _OUTW,), jnp.float32),
    mesh=plsc.VectorSubcoreMesh(core_axis_name="c", subcore_axis_name="s",
                                num_cores=_NC, num_subcores=_NS),
    compiler_params=pltpu.CompilerParams(needs_layout_passes=False),
    scratch_types=[
        pltpu.VMEM((_N,), jnp.float32),
        pltpu.VMEM((_N,), jnp.float32),
        pltpu.VMEM((_N,), jnp.float32),
        pltpu.VMEM((_N,), jnp.float32),
        pltpu.VMEM((_N,), jnp.float32),
        pltpu.VMEM((_N,), jnp.float32),
        pltpu.VMEM((_N,), jnp.float32),
        pltpu.VMEM((_N,), jnp.float32),
        pltpu.VMEM((max(_QPW, 128),), jnp.int32),
        pltpu.VMEM((_NQB * _S * _L,), jnp.float32),
        pltpu.VMEM((_NQB * _S * _L,), jnp.int32),
        pltpu.VMEM((_NQB * _S2PAD * _L,), jnp.float32),
        pltpu.VMEM((_NQB * _S2PAD * _L,), jnp.int32),
        pltpu.VMEM((_QPW * _OUTW,), jnp.float32),
    ],
)(_knn_body)


def _mlp_body(sc_ref, w1_ref, b1_ref, w2_ref, b2_ref, afc_ref, anv_ref,
              wfc_ref, wft_ref, bin_ref, wb_ref, bb_ref, wo_ref, bo_ref,
              out_ref):
    f32 = jnp.float32

    def dot(a, bm):
        return lax.dot_general(a, bm, (((1,), (0,)), ((), ())),
                               preferred_element_type=f32)
    sc = sc_ref[...]
    h = jnp.maximum(dot(sc, w1_ref[...]) + b1_ref[...], 0.0)
    feat = dot(h, w2_ref[...]) + b2_ref[...]
    fc = dot(sc, afc_ref[...])
    nv = dot(sc, anv_ref[...])
    s = jnp.maximum(dot(fc, wfc_ref[...]) + dot(feat, wft_ref[...])
                    + bin_ref[...], 0.0)
    for i in range(4):
        s = jnp.maximum(dot(s, wb_ref[i]) + bb_ref[i][None, :], 0.0) + s
    g = dot(s, wo_ref[...]) + bo_ref[...]
    diff = nv + g
    out_ref[...] = (jnp.sum(diff * diff) * f32(0.5 * 100.0 / (_B * _Q))
                    ).reshape(1, 1)


def kernel(pcl_noisy, pcl_clean, params, pnt_idx):
    f32 = jnp.float32
    noisy_flat = jnp.transpose(pcl_noisy, (0, 2, 1)).reshape(-1)
    clean_flat = jnp.transpose(pcl_clean, (0, 2, 1)).reshape(-1)
    sc = _knn_call(noisy_flat, clean_flat,
                   pnt_idx.astype(jnp.int32)).reshape(_B * _Q, _OUTW)

    p = params
    w1 = jnp.zeros((_OUTW, 64), f32).at[0:3].set(p['fW1'])
    b1 = p['fb1'].reshape(1, 64)
    w2 = p['fW2']
    b2 = p['fb2'].reshape(1, 128)
    afc = (jnp.zeros((_OUTW, _OUTW), f32)
           .at[0, 0].set(-1.0).at[1, 1].set(-1.0).at[2, 2].set(-1.0)
           .at[3, 0].set(1.0 / 32).at[4, 1].set(1.0 / 32)
           .at[5, 2].set(1.0 / 32))
    anv = (jnp.zeros((_OUTW, _OUTW), f32)
           .at[0, 0].set(1.0).at[1, 1].set(1.0).at[2, 2].set(1.0)
           .at[6, 0].set(-0.25).at[7, 1].set(-0.25).at[8, 2].set(-0.25))
    wfc = jnp.zeros((_OUTW, 128), f32).at[0:3].set(p['sWin'][0:3])
    wft = p['sWin'][3:]
    bin_ = p['sbin'].reshape(1, 128)
    wb = jnp.stack(p['sWb'])
    bb = jnp.stack(p['sbb'])
    wo = jnp.zeros((128, _OUTW), f32).at[:, 0:3].set(p['sWout'])
    bo = jnp.zeros((1, _OUTW), f32).at[0, 0:3].set(p['sbout'])

    loss = pl.pallas_call(
        _mlp_body,
        out_shape=jax.ShapeDtypeStruct((1, 1), f32),
    )(sc, w1, b1, w2, b2, afc, anv, wfc, wft, bin_, wb, bb, wo, bo)
    return loss[0, 0]


# speedup vs baseline: 1.4020x; 1.4020x over previous
import functools

import jax
import jax.numpy as jnp
from jax import lax
from jax.experimental import pallas as pl
from jax.experimental.pallas import tpu as pltpu
from jax.experimental.pallas import tpu_sc as plsc

_NC, _NS, _L = 2, 16, 16
_NW = _NC * _NS

_B, _N, _Q = 4, 10000, 512
_QPW = (_B * _Q) // _NW
_SLOTS = _Q // _QPW
_NCH = _N // _L
_GROUP = 5
_NGRP = _NCH // _GROUP
_NQB = 4
_S = 128
_S2 = 32
_S2PAD = _S2 + 4
_OUTW = 16

_BIG = 3e38


def _knn_body(noisy_hbm, clean_hbm, idx_hbm, out_hbm,
              px, py, pz, cx, cy, cz, sqn, sqc, idx_v,
              cand_d, cand_i, b2d, b2i, out_v):
    wid = lax.axis_index("s") * _NC + lax.axis_index("c")
    b = wid // _SLOTS
    slot = wid % _SLOTS
    base = b * 3 * _N
    pltpu.sync_copy(noisy_hbm.at[pl.ds(base, _N)], px)
    pltpu.sync_copy(noisy_hbm.at[pl.ds(base + _N, _N)], py)
    pltpu.sync_copy(noisy_hbm.at[pl.ds(base + 2 * _N, _N)], pz)
    pltpu.sync_copy(clean_hbm.at[pl.ds(base, _N)], cx)
    pltpu.sync_copy(clean_hbm.at[pl.ds(base + _N, _N)], cy)
    pltpu.sync_copy(clean_hbm.at[pl.ds(base + 2 * _N, _N)], cz)
    pltpu.sync_copy(idx_hbm.at[pl.ds(slot * _QPW, _QPW)],
                    idx_v.at[pl.ds(0, _QPW)])

    def sqinit(c, _):
        sl = pl.ds(c * _L, _L)
        vx = px[sl]
        vy = py[sl]
        vz = pz[sl]
        sqn[sl] = vx * vx + vy * vy + vz * vz
        vx = cx[sl]
        vy = cy[sl]
        vz = cz[sl]
        sqc[sl] = vx * vx + vy * vy + vz * vz
        return _
    lax.fori_loop(0, _NCH, sqinit, 0)

    iota = lax.iota(jnp.int32, _L)
    zf = jnp.zeros((_L,), jnp.float32)
    zi = jnp.zeros((_L,), jnp.int32)
    cbase = [qq * (_S * _L) for qq in range(_NQB)]
    climits = [cbase[qq] + (_S - 1) * _L + iota for qq in range(_NQB)]

    def _shuf(v, s):
        return v.at[iota ^ s].get(mode="promise_in_bounds")

    def vmaxs(v):
        for s in (8, 4, 2, 1):
            v = jnp.maximum(v, _shuf(v, s))
        return v

    def vadds(v):
        for s in (8, 4, 2, 1):
            v = v + _shuf(v, s)
        return v

    def scan4(xr, yr, zr, sq, tqx, tqy, tqz, depth):
        def grp(g, carry):
            m1s, m2s, offp, tbs = carry
            m1s, m2s, offp, tbs = list(m1s), list(m2s), list(offp), list(tbs)
            for k in range(_GROUP):
                c = g * _GROUP + k
                sl = pl.ds(c * _L, _L)
                vx = xr[sl]
                vy = yr[sl]
                vz = zr[sl]
                vs = sq[sl]
                ci = c * _L + iota
                for qq in range(_NQB):
                    d = (vs - vx * tqx[qq]
                         - vy * tqy[qq] - vz * tqz[qq])
                    if depth == 2:
                        m2s[qq] = jnp.minimum(m2s[qq],
                                              jnp.maximum(m1s[qq], d))
                    m1s[qq] = jnp.minimum(m1s[qq], d)
                    msk = d <= tbs[qq]
                    pos = jnp.minimum(offp[qq], climits[qq])
                    plsc.store_scatter(cand_d, [pos], d, mask=msk)
                    plsc.store_scatter(cand_i, [pos], ci, mask=msk)
                    offp[qq] = offp[qq] + jnp.where(msk, 16, 0)
            for qq in range(_NQB):
                tbs[qq] = vmaxs(m2s[qq] if depth == 2 else m1s[qq])
            return tuple(m1s), tuple(m2s), tuple(offp), tuple(tbs)

        big = jnp.full((_L,), _BIG, jnp.float32)
        init = ((big,) * _NQB, (big,) * _NQB,
                tuple(cbase[qq] + iota for qq in range(_NQB)), (big,) * _NQB)
        m1s, m2s, offp, _ = lax.fori_loop(0, _NGRP, grp, init)
        msrc = m2s if depth == 2 else m1s
        offls = tuple((offp[qq] - cbase[qq] - iota) >> 4
                      for qq in range(_NQB))
        return msrc, offls

    def finish4(msrcs, offls, xr, yr, zr, K):
        texs = [vmaxs(msrcs[qq]) for qq in range(_NQB)]
        rmax = jnp.maximum(jnp.maximum(offls[0], offls[1]),
                           jnp.maximum(offls[2], offls[3]))
        rows = jnp.minimum(jnp.max(rmax), _S)

        def shrink(r, off2s):
            off2s = list(off2s)
            for qq in range(_NQB):
                sl = pl.ds(qq * (_S * _L) + r * _L, _L)
                d = cand_d[sl]
                vi = cand_i[sl]
                msk = (d <= texs[qq]) & (offls[qq] > r)
                pos = (qq * (_S2PAD * _L)
                       + jnp.minimum(off2s[qq], _S2 - 1) * _L + iota)
                plsc.store_scatter(b2d, [pos], d, mask=msk)
                plsc.store_scatter(b2i, [pos], vi, mask=msk)
                off2s[qq] = off2s[qq] + msk.astype(jnp.int32)
            return tuple(off2s)
        off2s = lax.fori_loop(0, rows, shrink, (zi,) * _NQB)
        off2s = [jnp.minimum(o, _S2) for o in off2s]
        rows2 = jnp.max(jnp.maximum(jnp.maximum(off2s[0], off2s[1]),
                                    jnp.maximum(off2s[2], off2s[3])))

        big = jnp.full((_L,), _BIG, jnp.float32)
        if K == 32:
            def fold(r, carry):
                new = []
                for qq in range(_NQB):
                    lok, lov, hik, hiv = carry[qq]
                    sl = pl.ds(qq * (_S2PAD * _L) + r * _L, _L)
                    kd = jnp.where(off2s[qq] > r, b2d[sl], big)
                    sk, sv = plsc.sort_key_val(kd, b2i[sl])
                    rsk = _shuf(sk, 15)
                    rsv = _shuf(sv, 15)
                    m = hik <= rsk
                    wk = jnp.where(m, hik, rsk)
                    wv = jnp.where(m, hiv, rsv)
                    wk, wv = plsc.sort_key_val(wk, wv)
                    rwk = _shuf(wk, 15)
                    rwv = _shuf(wv, 15)
                    m2 = lok <= rwk
                    nlk = jnp.where(m2, lok, rwk)
                    nlv = jnp.where(m2, lov, rwv)
                    nhk = jnp.where(m2, rwk, lok)
                    nhv = jnp.where(m2, rwv, lov)
                    nlk, nlv = plsc.sort_key_val(nlk, nlv)
                    nhk, nhv = plsc.sort_key_val(nhk, nhv)
                    new.append((nlk, nlv, nhk, nhv))
                return tuple(new)
            st = lax.fori_loop(0, rows2, fold, ((big, zi, big, zi),) * _NQB)
            sums = []
            for qq in range(_NQB):
                _, lov, _, hiv = st[qq]
                gx = plsc.load_gather(xr, [lov]) + plsc.load_gather(xr, [hiv])
                gy = plsc.load_gather(yr, [lov]) + plsc.load_gather(yr, [hiv])
                gz = plsc.load_gather(zr, [lov]) + plsc.load_gather(zr, [hiv])
                sums.append((vadds(gx), vadds(gy), vadds(gz)))
            return sums
        else:
            def fold(r, carry):
                new = []
                for qq in range(_NQB):
                    lok, lov = carry[qq]
                    sl = pl.ds(qq * (_S2PAD * _L) + r * _L, _L)
                    kd = jnp.where(off2s[qq] > r, b2d[sl], big)
                    sk, sv = plsc.sort_key_val(kd, b2i[sl])
                    rsk = _shuf(sk, 15)
                    rsv = _shuf(sv, 15)
                    m = lok <= rsk
                    wk = jnp.where(m, lok, rsk)
                    wv = jnp.where(m, lov, rsv)
                    lok, lov = plsc.sort_key_val(wk, wv)
                    new.append((lok, lov))
                return tuple(new)
            st = lax.fori_loop(0, rows2, fold, ((big, zi),) * _NQB)
            mk = iota < K
            sums = []
            for qq in range(_NQB):
                _, lov = st[qq]
                gx = plsc.load_gather(xr, [lov], mask=mk)
                gy = plsc.load_gather(yr, [lov], mask=mk)
                gz = plsc.load_gather(zr, [lov], mask=mk)
                sums.append((vadds(jnp.where(mk, gx, zf)),
                             vadds(jnp.where(mk, gy, zf)),
                             vadds(jnp.where(mk, gz, zf))))
            return sums

    def qgroup(jg, carry):
        j0 = jg * _NQB
        qxs, qys, qzs = [], [], []
        for qq in range(_NQB):
            qidx = plsc.load_gather(
                idx_v, [jnp.full((_L,), j0 + qq, jnp.int32)])
            qxs.append(plsc.load_gather(px, [qidx]))
            qys.append(plsc.load_gather(py, [qidx]))
            qzs.append(plsc.load_gather(pz, [qidx]))
        tqx = [qxs[qq] + qxs[qq] for qq in range(_NQB)]
        tqy = [qys[qq] + qys[qq] for qq in range(_NQB)]
        tqz = [qzs[qq] + qzs[qq] for qq in range(_NQB)]
        msrc, offls = scan4(px, py, pz, sqn, tqx, tqy, tqz, 2)
        nsum = finish4(msrc, offls, px, py, pz, 32)
        msrc, offls = scan4(cx, cy, cz, sqc, tqx, tqy, tqz, 1)
        esum = finish4(msrc, offls, cx, cy, cz, 4)
        for qq in range(_NQB):
            out = jnp.where(iota == 0, qxs[qq], zf)
            out = jnp.where(iota == 1, qys[qq], out)
            out = jnp.where(iota == 2, qzs[qq], out)
            out = jnp.where(iota == 3, nsum[qq][0], out)
            out = jnp.where(iota == 4, nsum[qq][1], out)
            out = jnp.where(iota == 5, nsum[qq][2], out)
            out = jnp.where(iota == 6, esum[qq][0], out)
            out = jnp.where(iota == 7, esum[qq][1], out)
            out = jnp.where(iota == 8, esum[qq][2], out)
            out_v[pl.ds((j0 + qq) * _OUTW, _OUTW)] = out
        return carry
    lax.fori_loop(0, _QPW // _NQB, qgroup, 0)
    pltpu.sync_copy(out_v, out_hbm.at[pl.ds(wid * _QPW * _OUTW, _QPW * _OUTW)])


_knn_call = functools.partial(
    pl.kernel,
    out_type=jax.ShapeDtypeStruct((_NW * _QPW * _OUTW,), jnp.float32),
    mesh=plsc.VectorSubcoreMesh(core_axis_name="c", subcore_axis_name="s",
                                num_cores=_NC, num_subcores=_NS),
    compiler_params=pltpu.CompilerParams(needs_layout_passes=False),
    scratch_types=[
        pltpu.VMEM((_N,), jnp.float32),
        pltpu.VMEM((_N,), jnp.float32),
        pltpu.VMEM((_N,), jnp.float32),
        pltpu.VMEM((_N,), jnp.float32),
        pltpu.VMEM((_N,), jnp.float32),
        pltpu.VMEM((_N,), jnp.float32),
        pltpu.VMEM((_N,), jnp.float32),
        pltpu.VMEM((_N,), jnp.float32),
        pltpu.VMEM((max(_QPW, 128),), jnp.int32),
        pltpu.VMEM((_NQB * _S * _L,), jnp.float32),
        pltpu.VMEM((_NQB * _S * _L,), jnp.int32),
        pltpu.VMEM((_NQB * _S2PAD * _L,), jnp.float32),
        pltpu.VMEM((_NQB * _S2PAD * _L,), jnp.int32),
        pltpu.VMEM((_QPW * _OUTW,), jnp.float32),
    ],
)(_knn_body)


def _mlp_body(sc_ref, w1_ref, b1_ref, w2_ref, b2_ref, afc_ref, anv_ref,
              wfc_ref, wft_ref, bin_ref, wb_ref, bb_ref, wo_ref, bo_ref,
              out_ref):
    f32 = jnp.float32

    def dot(a, bm):
        return lax.dot_general(a, bm, (((1,), (0,)), ((), ())),
                               preferred_element_type=f32)
    sc = sc_ref[...]
    h = jnp.maximum(dot(sc, w1_ref[...]) + b1_ref[...], 0.0)
    feat = dot(h, w2_ref[...]) + b2_ref[...]
    fc = dot(sc, afc_ref[...])
    nv = dot(sc, anv_ref[...])
    s = jnp.maximum(dot(fc, wfc_ref[...]) + dot(feat, wft_ref[...])
                    + bin_ref[...], 0.0)
    for i in range(4):
        s = jnp.maximum(dot(s, wb_ref[i]) + bb_ref[i][None, :], 0.0) + s
    g = dot(s, wo_ref[...]) + bo_ref[...]
    diff = nv + g
    out_ref[...] = (jnp.sum(diff * diff) * f32(0.5 * 100.0 / (_B * _Q))
                    ).reshape(1, 1)


def kernel(pcl_noisy, pcl_clean, params, pnt_idx):
    f32 = jnp.float32
    noisy_flat = jnp.transpose(pcl_noisy, (0, 2, 1)).reshape(-1)
    clean_flat = jnp.transpose(pcl_clean, (0, 2, 1)).reshape(-1)
    sc = _knn_call(noisy_flat, clean_flat,
                   pnt_idx.astype(jnp.int32)).reshape(_B * _Q, _OUTW)

    p = params
    w1 = jnp.zeros((_OUTW, 64), f32).at[0:3].set(p['fW1'])
    b1 = p['fb1'].reshape(1, 64)
    w2 = p['fW2']
    b2 = p['fb2'].reshape(1, 128)
    afc = (jnp.zeros((_OUTW, _OUTW), f32)
           .at[0, 0].set(-1.0).at[1, 1].set(-1.0).at[2, 2].set(-1.0)
           .at[3, 0].set(1.0 / 32).at[4, 1].set(1.0 / 32)
           .at[5, 2].set(1.0 / 32))
    anv = (jnp.zeros((_OUTW, _OUTW), f32)
           .at[0, 0].set(1.0).at[1, 1].set(1.0).at[2, 2].set(1.0)
           .at[6, 0].set(-0.25).at[7, 1].set(-0.25).at[8, 2].set(-0.25))
    wfc = jnp.zeros((_OUTW, 128), f32).at[0:3].set(p['sWin'][0:3])
    wft = p['sWin'][3:]
    bin_ = p['sbin'].reshape(1, 128)
    wb = jnp.stack(p['sWb'])
    bb = jnp.stack(p['sbb'])
    wo = jnp.zeros((128, _OUTW), f32).at[:, 0:3].set(p['sWout'])
    bo = jnp.zeros((1, _OUTW), f32).at[0, 0:3].set(p['sbout'])

    loss = pl.pallas_call(
        _mlp_body,
        out_shape=jax.ShapeDtypeStruct((1, 1), f32),
    )(sc, w1, b1, w2, b2, afc, anv, wfc, wft, bin_, wb, bb, wo, bo)
    return loss[0, 0]
